# speedup vs baseline: 1.2292x; 1.2292x over previous
import functools

import jax
import jax.numpy as jnp
from jax import lax
from jax.experimental import pallas as pl
from jax.experimental.pallas import tpu as pltpu
from jax.experimental.pallas import tpu_sc as plsc

D = 128
B = 16384
L = 16
NC, NS = 2, 16
NW = NC * NS
BPW = B // NW
CHUNK = 128
NCHUNK = BPW // CHUNK


def _build():
    mesh = plsc.VectorSubcoreMesh(core_axis_name="c", subcore_axis_name="s")

    @functools.partial(
        pl.kernel,
        mesh=mesh,
        out_type=jax.ShapeDtypeStruct((B, D), jnp.float32),
        scratch_types=[
            pltpu.VMEM((NCHUNK, CHUNK), jnp.int32),
            pltpu.VMEM((CHUNK, D), jnp.float32),
            pltpu.VMEM((CHUNK, D), jnp.float32),
            pltpu.SemaphoreType.DMA,
            pltpu.SemaphoreType.DMA,
        ],
    )
    def gather_sigmoid(table_hbm, idx_hbm, out_hbm, idx_v, buf0, buf1,
                       sem0, sem1):
        wid = lax.axis_index("s") * NC + lax.axis_index("c")
        base = wid * BPW
        pltpu.sync_copy(idx_hbm.at[wid], idx_v)

        bufs = (buf0, buf1)
        sems = (sem0, sem1)

        def fire(j):
            return pltpu.async_copy(
                table_hbm.at[idx_v.at[j]], bufs[j % 2], sems[j % 2])

        def sigmoid_inplace(buf):
            def row_body(r, carry):
                for k in range(D // L):
                    sl = pl.ds(k * L, L)
                    x = buf[r, sl]
                    buf[r, sl] = 1.0 / (1.0 + jnp.exp(-x))
                return carry
            lax.fori_loop(0, CHUNK, row_body, 0)

        handles = [None] * NCHUNK
        handles[0] = fire(0)
        for j in range(NCHUNK):
            handles[j].wait()
            if j + 1 < NCHUNK:
                handles[j + 1] = fire(j + 1)
            buf = bufs[j % 2]
            sigmoid_inplace(buf)
            pltpu.sync_copy(buf, out_hbm.at[pl.ds(base + j * CHUNK, CHUNK)])

    return gather_sigmoid


_GATHER_SIGMOID = _build()


def kernel(slice_num, optimized_array):
    idx = slice_num.reshape(NW, NCHUNK, CHUNK)
    return _GATHER_SIGMOID(optimized_array, idx)


# speedup vs baseline: 1.3471x; 1.0959x over previous
import functools

import jax
import jax.numpy as jnp
from jax import lax
from jax.experimental import pallas as pl
from jax.experimental.pallas import tpu as pltpu
from jax.experimental.pallas import tpu_sc as plsc

D = 128
B = 16384
L = 16
NC, NS = 2, 16
NW = NC * NS
BPW = B // NW
CHUNK = 128
NCHUNK = BPW // CHUNK


def _build():
    mesh = plsc.VectorSubcoreMesh(core_axis_name="c", subcore_axis_name="s")

    @functools.partial(
        pl.kernel,
        mesh=mesh,
        out_type=jax.ShapeDtypeStruct((B, D), jnp.float32),
        scratch_types=(
            [pltpu.VMEM((NCHUNK, CHUNK), jnp.int32)]
            + [pltpu.VMEM((CHUNK, D), jnp.float32)] * NCHUNK
            + [pltpu.SemaphoreType.DMA] * (2 * NCHUNK)
        ),
    )
    def gather_sigmoid(table_hbm, idx_hbm, out_hbm, idx_v, *scr):
        bufs = scr[:NCHUNK]
        gsems = scr[NCHUNK:2 * NCHUNK]
        wsems = scr[2 * NCHUNK:]
        wid = lax.axis_index("s") * NC + lax.axis_index("c")
        base = wid * BPW
        pltpu.sync_copy(idx_hbm.at[wid], idx_v)

        gathers = [
            pltpu.async_copy(table_hbm.at[idx_v.at[j]], bufs[j], gsems[j])
            for j in range(NCHUNK)
        ]

        def sigmoid_inplace(buf):
            def row_body(i, carry):
                r = i * 2
                for rr in range(2):
                    for k in range(D // L):
                        sl = pl.ds(k * L, L)
                        x = buf[r + rr, sl]
                        buf[r + rr, sl] = 1.0 / (1.0 + jnp.exp(-x))
                return carry
            lax.fori_loop(0, CHUNK // 2, row_body, 0)

        writes = []
        for j in range(NCHUNK):
            gathers[j].wait()
            sigmoid_inplace(bufs[j])
            writes.append(pltpu.async_copy(
                bufs[j], out_hbm.at[pl.ds(base + j * CHUNK, CHUNK)],
                wsems[j]))
        for w in writes:
            w.wait()

    return gather_sigmoid


_GATHER_SIGMOID = _build()


def kernel(slice_num, optimized_array):
    idx = slice_num.reshape(NW, NCHUNK, CHUNK)
    return _GATHER_SIGMOID(optimized_array, idx)


# speedup vs baseline: 1.3505x; 1.0025x over previous
import functools

import jax
import jax.numpy as jnp
from jax import lax
from jax.experimental import pallas as pl
from jax.experimental.pallas import tpu as pltpu
from jax.experimental.pallas import tpu_sc as plsc

D = 128
B = 16384
L = 16
NC, NS = 2, 16
NW = NC * NS
BPW = B // NW
CHUNK = 128
NCHUNK = BPW // CHUNK


def _build():
    mesh = plsc.VectorSubcoreMesh(core_axis_name="c", subcore_axis_name="s")

    @functools.partial(
        pl.kernel,
        mesh=mesh,
        out_type=jax.ShapeDtypeStruct((B, D), jnp.float32),
        scratch_types=(
            [pltpu.VMEM((NCHUNK, CHUNK), jnp.int32)]
            + [pltpu.VMEM((CHUNK, D), jnp.float32)] * NCHUNK
            + [pltpu.SemaphoreType.DMA] * (2 * NCHUNK)
        ),
    )
    def gather_sigmoid(table_hbm, idx_hbm, out_hbm, idx_v, *scr):
        bufs = scr[:NCHUNK]
        gsems = scr[NCHUNK:2 * NCHUNK]
        wsems = scr[2 * NCHUNK:]
        wid = lax.axis_index("s") * NC + lax.axis_index("c")
        base = wid * BPW
        pltpu.sync_copy(idx_hbm.at[wid], idx_v)

        gathers = [
            pltpu.async_copy(table_hbm.at[idx_v.at[j]], bufs[j], gsems[j])
            for j in range(NCHUNK)
        ]

        UNROLL = 4

        def sigmoid_inplace(buf):
            def row_body(i, carry):
                r = i * UNROLL
                for rr in range(UNROLL):
                    for k in range(D // L):
                        sl = pl.ds(k * L, L)
                        x = buf[r + rr, sl]
                        buf[r + rr, sl] = 1.0 / (1.0 + jnp.exp(-x))
                return carry
            lax.fori_loop(0, CHUNK // UNROLL, row_body, 0)

        writes = []
        for j in range(NCHUNK):
            gathers[j].wait()
            sigmoid_inplace(bufs[j])
            writes.append(pltpu.async_copy(
                bufs[j], out_hbm.at[pl.ds(base + j * CHUNK, CHUNK)],
                wsems[j]))
        for w in writes:
            w.wait()

    return gather_sigmoid


_GATHER_SIGMOID = _build()


def kernel(slice_num, optimized_array):
    idx = slice_num.reshape(NW, NCHUNK, CHUNK)
    return _GATHER_SIGMOID(optimized_array, idx)


# speedup vs baseline: 1.3744x; 1.0177x over previous
import functools

import jax
import jax.numpy as jnp
from jax import lax
from jax.experimental import pallas as pl
from jax.experimental.pallas import tpu as pltpu
from jax.experimental.pallas import tpu_sc as plsc

D = 128
B = 16384
L = 16
NC, NS = 2, 16
NW = NC * NS
BPW = B // NW
CHUNK = 128
NCHUNK = BPW // CHUNK


def _build():
    mesh = plsc.VectorSubcoreMesh(core_axis_name="c", subcore_axis_name="s")

    @functools.partial(
        pl.kernel,
        mesh=mesh,
        out_type=jax.ShapeDtypeStruct((B, D), jnp.float32),
        scratch_types=(
            [pltpu.VMEM((NCHUNK, CHUNK), jnp.int32)]
            + [pltpu.VMEM((CHUNK, D), jnp.float32)] * NCHUNK
            + [pltpu.SemaphoreType.DMA] * (2 * NCHUNK)
        ),
    )
    def gather_sigmoid(table_hbm, idx_hbm, out_hbm, idx_v, *scr):
        bufs = scr[:NCHUNK]
        gsems = scr[NCHUNK:2 * NCHUNK]
        wsems = scr[2 * NCHUNK:]
        wid = lax.axis_index("s") * NC + lax.axis_index("c")
        base = wid * BPW
        pltpu.sync_copy(idx_hbm.at[wid], idx_v)

        gathers = [
            pltpu.async_copy(table_hbm.at[idx_v.at[j]], bufs[j], gsems[j])
            for j in range(NCHUNK)
        ]

        def sigmoid_inplace(buf):
            @plsc.parallel_loop(0, CHUNK, step=1, unroll=4)
            def _rows(r):
                for k in range(D // L):
                    sl = pl.ds(k * L, L)
                    x = buf[r, sl]
                    buf[r, sl] = 1.0 / (1.0 + jnp.exp(-x))

        writes = []
        for j in range(NCHUNK):
            gathers[j].wait()
            sigmoid_inplace(bufs[j])
            writes.append(pltpu.async_copy(
                bufs[j], out_hbm.at[pl.ds(base + j * CHUNK, CHUNK)],
                wsems[j]))
        for w in writes:
            w.wait()

    return gather_sigmoid


_GATHER_SIGMOID = _build()


def kernel(slice_num, optimized_array):
    idx = slice_num.reshape(NW, NCHUNK, CHUNK)
    return _GATHER_SIGMOID(optimized_array, idx)
